# Initial kernel scaffold; baseline (speedup 1.0000x reference)
import jax
import jax.numpy as jnp
from jax.experimental import pallas as pl


def kernel(item_ids, action_ids, item_table, actn_table):
    raise NotImplementedError("write your pallas kernel here")


# speedup vs baseline: 2.2764x; 2.2764x over previous
import functools

import jax
import jax.numpy as jnp
from jax import lax
from jax.experimental import pallas as pl
from jax.experimental.pallas import tpu as pltpu
from jax.experimental.pallas import tpu_sc as plsc

ITEM_DIM = 64
ACTN_DIM = 32
OUT_DIM = ITEM_DIM + ACTN_DIM
PAD_DIM = 128
IDXW = 128
K = 4
CHUNK = K * IDXW
LANES = 16


def _make_kernel(n_total: int):
    info = plsc.get_sparse_core_info()
    nw = info.num_cores * info.num_subcores
    per_w = n_total // nw
    n_chunks = per_w // CHUNK

    mesh = plsc.VectorSubcoreMesh(core_axis_name="c", subcore_axis_name="s")

    @functools.partial(
        pl.kernel,
        mesh=mesh,
        compiler_params=pltpu.CompilerParams(needs_layout_passes=False),
        out_type=jax.ShapeDtypeStruct((n_total, PAD_DIM), jnp.float32),
        scratch_types=[
            pltpu.VMEM((K, IDXW), jnp.int32),
            pltpu.VMEM((K, IDXW), jnp.int32),
            pltpu.VMEM((CHUNK, PAD_DIM), jnp.float32),
            pltpu.VMEM((4, ACTN_DIM), jnp.float32),
            pltpu.SemaphoreType.DMA,
        ],
    )
    def emb_kernel(item_idx, actn_idx, item_tab, actn_tab, out,
                   iidx_v, aidx_v, rows_v, atab_v, sem):
        wid = lax.axis_index("s") * info.num_cores + lax.axis_index("c")
        w_row0 = wid * (per_w // IDXW)

        pltpu.sync_copy(actn_tab, atab_v)

        lane = lax.iota(jnp.int32, LANES)
        groups_per_row = IDXW // LANES

        def chunk_body(i, _):
            row0 = w_row0 + i * K
            pltpu.sync_copy(item_idx.at[pl.ds(row0, K)], iidx_v)
            pltpu.sync_copy(actn_idx.at[pl.ds(row0, K)], aidx_v)
            copies = [
                pltpu.async_copy(
                    item_tab.at[iidx_v.at[k]],
                    rows_v.at[pl.ds(k * IDXW, IDXW)], sem)
                for k in range(K)
            ]
            for c in copies:
                c.wait()

            def group_body(g, _):
                k = g // groups_per_row
                c0 = (g % groups_per_row) * LANES
                aid = aidx_v.at[k][pl.ds(c0, LANES)]
                row_idx = g * LANES + lane
                for j in range(ACTN_DIM):
                    col = plsc.load_gather(
                        atab_v, [aid, jnp.full((LANES,), j, jnp.int32)])
                    plsc.store_scatter(
                        rows_v,
                        [row_idx, jnp.full((LANES,), ITEM_DIM + j, jnp.int32)],
                        col)
                return ()

            lax.fori_loop(0, CHUNK // LANES, group_body, ())

            base = wid * per_w + i * CHUNK
            pltpu.sync_copy(rows_v, out.at[pl.ds(base, CHUNK)])
            return ()

        lax.fori_loop(0, n_chunks, chunk_body, ())

    return emb_kernel


def kernel(item_ids, action_ids, item_table, actn_table):
    b, l = item_ids.shape
    n_total = b * l
    iidx = item_ids.reshape(n_total // IDXW, IDXW).astype(jnp.int32)
    aidx = action_ids.reshape(n_total // IDXW, IDXW).astype(jnp.int32)
    tab128 = jnp.pad(item_table, ((0, 0), (0, PAD_DIM - ITEM_DIM)))
    out128 = _make_kernel(n_total)(iidx, aidx, tab128, actn_table)
    return out128[:, :OUT_DIM].reshape(b, l, OUT_DIM)


# speedup vs baseline: 2.3897x; 1.0498x over previous
import functools

import jax
import jax.numpy as jnp
from jax import lax
from jax.experimental import pallas as pl
from jax.experimental.pallas import tpu as pltpu
from jax.experimental.pallas import tpu_sc as plsc

ITEM_DIM = 64
ACTN_DIM = 32
OUT_DIM = ITEM_DIM + ACTN_DIM
PAD_DIM = 128
IDXW = 128
K = 2
CHUNK = K * IDXW
LANES = 16
NBUF = 2


def _make_kernel(n_total: int):
    info = plsc.get_sparse_core_info()
    nw = info.num_cores * info.num_subcores
    per_w = n_total // nw
    n_chunks = per_w // CHUNK

    mesh = plsc.VectorSubcoreMesh(core_axis_name="c", subcore_axis_name="s")

    @functools.partial(
        pl.kernel,
        mesh=mesh,
        compiler_params=pltpu.CompilerParams(needs_layout_passes=False),
        out_type=jax.ShapeDtypeStruct((n_total, PAD_DIM), jnp.float32),
        scratch_types=[
            pltpu.VMEM((NBUF, K, IDXW), jnp.int32),
            pltpu.VMEM((NBUF, K, IDXW), jnp.int32),
            pltpu.VMEM((NBUF, CHUNK, PAD_DIM), jnp.float32),
            pltpu.VMEM((4, ACTN_DIM), jnp.float32),
            pltpu.SemaphoreType.DMA,
            pltpu.SemaphoreType.DMA,
        ],
    )
    def emb_kernel(item_idx, actn_idx, item_tab, actn_tab, out,
                   iidx_v, aidx_v, rows_v, atab_v, gsem, osem):
        wid = lax.axis_index("s") * info.num_cores + lax.axis_index("c")
        w_row0 = wid * (per_w // IDXW)
        w_base = wid * per_w

        pltpu.sync_copy(actn_tab, atab_v)

        lane = lax.iota(jnp.int32, LANES)
        groups_per_row = IDXW // LANES

        def fire(i, slot):
            row0 = w_row0 + i * K
            pltpu.sync_copy(item_idx.at[pl.ds(row0, K)], iidx_v.at[slot])
            pltpu.sync_copy(actn_idx.at[pl.ds(row0, K)], aidx_v.at[slot])
            for k in range(K):
                pltpu.async_copy(
                    item_tab.at[iidx_v.at[slot, k]],
                    rows_v.at[slot, pl.ds(k * IDXW, IDXW)], gsem)

        def gather_wait(slot):
            for k in range(K):
                pltpu.make_async_copy(
                    item_tab.at[iidx_v.at[slot, k]],
                    rows_v.at[slot, pl.ds(k * IDXW, IDXW)], gsem).wait()

        def fill_action(slot):
            def group_body(g, _):
                k = g // groups_per_row
                c0 = (g % groups_per_row) * LANES
                aid = aidx_v.at[slot, k][pl.ds(c0, LANES)]
                row_idx = g * LANES + lane
                for j in range(ACTN_DIM):
                    col = plsc.load_gather(
                        atab_v, [aid, jnp.full((LANES,), j, jnp.int32)])
                    plsc.store_scatter(
                        rows_v.at[slot],
                        [row_idx, jnp.full((LANES,), ITEM_DIM + j, jnp.int32)],
                        col)
                return ()
            lax.fori_loop(0, CHUNK // LANES, group_body, ())

        def out_copy(i, slot):
            return pltpu.make_async_copy(
                rows_v.at[slot],
                out.at[pl.ds(w_base + i * CHUNK, CHUNK)], osem)

        fire(0, 0)

        def chunk_body(i, _):
            slot = lax.rem(i, NBUF)
            nxt = lax.rem(i + 1, NBUF)
            @pl.when(i >= 1)
            def _():
                out_copy(i - 1, nxt).wait()

            @pl.when(i + 1 < n_chunks)
            def _():
                fire(i + 1, nxt)

            gather_wait(slot)
            fill_action(slot)
            out_copy(i, slot).start()
            return ()

        lax.fori_loop(0, n_chunks, chunk_body, ())
        out_copy(n_chunks - 1, lax.rem(n_chunks - 1, NBUF)).wait()

    return emb_kernel


def kernel(item_ids, action_ids, item_table, actn_table):
    b, l = item_ids.shape
    n_total = b * l
    iidx = item_ids.reshape(n_total // IDXW, IDXW).astype(jnp.int32)
    aidx = action_ids.reshape(n_total // IDXW, IDXW).astype(jnp.int32)
    tab128 = jnp.pad(item_table, ((0, 0), (0, PAD_DIM - ITEM_DIM)))
    out128 = _make_kernel(n_total)(iidx, aidx, tab128, actn_table)
    return out128[:, :OUT_DIM].reshape(b, l, OUT_DIM)


# speedup vs baseline: 2.6150x; 1.0943x over previous
import functools

import jax
import jax.numpy as jnp
from jax import lax
from jax.experimental import pallas as pl
from jax.experimental.pallas import tpu as pltpu
from jax.experimental.pallas import tpu_sc as plsc

ITEM_DIM = 64
ACTN_DIM = 32
OUT_DIM = ITEM_DIM + ACTN_DIM
PAD_DIM = 128
UNIT = 128
LANES = 16
NBUF = 4
LAG = 2


def _make_kernel(batch: int, hist: int):
    n_total = batch * hist
    info = plsc.get_sparse_core_info()
    nw = info.num_cores * info.num_subcores
    b_per_w = batch // nw
    per_w = n_total // nw
    n_units = per_w // UNIT

    mesh = plsc.VectorSubcoreMesh(core_axis_name="c", subcore_axis_name="s")

    @functools.partial(
        pl.kernel,
        mesh=mesh,
        compiler_params=pltpu.CompilerParams(needs_layout_passes=False),
        out_type=jax.ShapeDtypeStruct((n_total, PAD_DIM), jnp.float32),
        scratch_types=[
            pltpu.VMEM((hist, b_per_w), jnp.int32),
            pltpu.VMEM((hist, b_per_w), jnp.int32),
            pltpu.VMEM((NBUF, UNIT), jnp.int32),
            pltpu.VMEM((NBUF, UNIT), jnp.int32),
            pltpu.VMEM((NBUF, UNIT, PAD_DIM), jnp.float32),
            pltpu.VMEM((4, ACTN_DIM), jnp.float32),
            pltpu.SemaphoreType.DMA,
            pltpu.SemaphoreType.DMA,
        ],
    )
    def emb_kernel(idsT, aidsT, item_tab, actn_tab, out,
                   ids_blk, aids_blk, iidx_v, aidx_v, rows_v, atab_v,
                   gsem, osem):
        wid = lax.axis_index("s") * info.num_cores + lax.axis_index("c")
        b0 = wid * b_per_w
        w_base = wid * per_w

        pltpu.sync_copy(idsT.at[:, pl.ds(b0, b_per_w)], ids_blk)
        pltpu.sync_copy(aidsT.at[:, pl.ds(b0, b_per_w)], aids_blk)
        pltpu.sync_copy(actn_tab, atab_v)

        lane = lax.iota(jnp.int32, LANES)
        hist_c = jnp.int32(hist)

        def build_idx(u, slot):
            n0 = u * UNIT
            for g in range(UNIT // LANES):
                n = n0 + g * LANES + lane
                b_loc = n // hist_c
                l = n - b_loc * hist_c
                ids = plsc.load_gather(ids_blk, [l, b_loc])
                aids = plsc.load_gather(aids_blk, [l, b_loc])
                iidx_v.at[slot][pl.ds(g * LANES, LANES)] = ids
                aidx_v.at[slot][pl.ds(g * LANES, LANES)] = aids

        def gather(u, slot):
            return pltpu.make_async_copy(
                item_tab.at[iidx_v.at[slot]], rows_v.at[slot], gsem)

        def fill_action(slot):
            for g in range(UNIT // LANES):
                aid = aidx_v.at[slot][pl.ds(g * LANES, LANES)]
                row_idx = g * LANES + lane
                for j in range(ACTN_DIM):
                    col = plsc.load_gather(
                        atab_v, [aid, jnp.full((LANES,), j, jnp.int32)])
                    plsc.store_scatter(
                        rows_v.at[slot],
                        [row_idx, jnp.full((LANES,), ITEM_DIM + j, jnp.int32)],
                        col)

        def out_copy(u, slot):
            return pltpu.make_async_copy(
                rows_v.at[slot],
                out.at[pl.ds(w_base + u * UNIT, UNIT)], osem)

        def step(u, _):
            slot = lax.rem(u, NBUF)

            @pl.when(u < n_units)
            def _():
                @pl.when(u >= NBUF)
                def _():
                    out_copy(u - NBUF, slot).wait()
                build_idx(u, slot)
                gather(u, slot).start()

            @pl.when(u >= LAG)
            def _():
                v = u - LAG
                vslot = lax.rem(v, NBUF)
                gather(v, vslot).wait()
                fill_action(vslot)
                out_copy(v, vslot).start()
            return ()

        lax.fori_loop(0, n_units + LAG, step, ())
        for t in range(NBUF):
            u = n_units - NBUF + t
            out_copy(u, lax.rem(jnp.int32(u), NBUF)).wait()

    return emb_kernel


def kernel(item_ids, action_ids, item_table, actn_table):
    b, l = item_ids.shape
    idsT = item_ids.astype(jnp.int32).T
    aidsT = action_ids.astype(jnp.int32).T
    tab128 = jnp.pad(item_table, ((0, 0), (0, PAD_DIM - ITEM_DIM)))
    out128 = _make_kernel(b, l)(idsT, aidsT, tab128, actn_table)
    return out128[:, :OUT_DIM].reshape(b, l, OUT_DIM)
